# Initial kernel scaffold; baseline (speedup 1.0000x reference)
import jax
import jax.numpy as jnp
from jax.experimental import pallas as pl


def kernel(x, gating_output, w1_q, w2_q, w1_scale, w2_scale, a1_scale, a2_scale):
    raise NotImplementedError("write your pallas kernel here")


# speedup vs baseline: 4.0935x; 4.0935x over previous
import jax
import jax.numpy as jnp
from jax.experimental import pallas as pl
from jax.experimental.pallas import tpu as pltpu

E = 64
TOPK = 2
DMODEL = 1024
DFF = 512
T = 64
FP8_MAX = 448.0


def _moe_body(x_ref, g_ref, w1_ref, w2_ref, s1_ref, s2_ref, a1_ref, a2_ref,
              out_ref):
    e = pl.program_id(0)

    probs = jax.nn.softmax(g_ref[...], axis=-1)
    p1 = jnp.max(probs, axis=-1)
    i1 = jnp.argmax(probs, axis=-1)
    col = jax.lax.broadcasted_iota(jnp.int32, (T, E), 1)
    probs2 = jnp.where(col == i1[:, None], -1.0, probs)
    p2 = jnp.max(probs2, axis=-1)
    i2 = jnp.argmax(probs2, axis=-1)
    wt = jnp.where(i1 == e, p1, jnp.where(i2 == e, p2, 0.0)) / (p1 + p2)

    a1 = a1_ref[0]
    a2 = a2_ref[0]
    xq = jnp.clip(x_ref[...] / a1, -FP8_MAX, FP8_MAX)
    h = jax.lax.dot_general(xq, w1_ref[0], (((1,), (1,)), ((), ())),
                            preferred_element_type=jnp.float32)
    h = h * (a1 * s1_ref[e])
    gate = h[:, :DFF]
    up = h[:, DFF:]
    act = gate * jax.nn.sigmoid(gate) * up
    aq = jnp.clip(act / a2, -FP8_MAX, FP8_MAX)
    oe = jax.lax.dot_general(aq, w2_ref[0], (((1,), (1,)), ((), ())),
                             preferred_element_type=jnp.float32)
    oe = oe * (a2 * s2_ref[e])
    contrib = wt[:, None] * oe

    @pl.when(e == 0)
    def _init():
        out_ref[...] = jnp.zeros_like(out_ref)

    out_ref[...] += contrib


def kernel(x, gating_output, w1_q, w2_q, w1_scale, w2_scale, a1_scale,
           a2_scale):
    s1 = w1_scale.reshape(E)
    s2 = w2_scale.reshape(E)
    a1 = a1_scale.reshape(1)
    a2 = a2_scale.reshape(1)
    return pl.pallas_call(
        _moe_body,
        grid=(E,),
        in_specs=[
            pl.BlockSpec((T, DMODEL), lambda e: (0, 0)),
            pl.BlockSpec((T, E), lambda e: (0, 0)),
            pl.BlockSpec((1, 2 * DFF, DMODEL), lambda e: (e, 0, 0)),
            pl.BlockSpec((1, DMODEL, DFF), lambda e: (e, 0, 0)),
            pl.BlockSpec(memory_space=pltpu.SMEM),
            pl.BlockSpec(memory_space=pltpu.SMEM),
            pl.BlockSpec(memory_space=pltpu.SMEM),
            pl.BlockSpec(memory_space=pltpu.SMEM),
        ],
        out_specs=pl.BlockSpec((T, DMODEL), lambda e: (0, 0)),
        out_shape=jax.ShapeDtypeStruct((T, DMODEL), jnp.float32),
    )(x, gating_output, w1_q, w2_q, s1, s2, a1, a2)
